# Initial kernel scaffold; baseline (speedup 1.0000x reference)
import jax
import jax.numpy as jnp
from jax.experimental import pallas as pl


def kernel(x, edge_index, W1_l, b1_l, W1_r, W2_l, b2_l, W2_r):
    raise NotImplementedError("write your pallas kernel here")


# speedup vs baseline: 5.1174x; 5.1174x over previous
import functools

import jax
import jax.numpy as jnp
from jax import lax
from jax.experimental import pallas as pl
from jax.experimental.pallas import tpu as pltpu
from jax.experimental.pallas import tpu_sc as plsc

N = 10000
E = 320000
D = 128

_INFO = plsc.get_sparse_core_info()
NC = _INFO.num_cores
NS = _INFO.num_subcores
NW = NC * NS
NP = 10240
EPW = E // NW
CHUNK = 80
NCHUNK = EPW // CHUNK
RPT = NP // NS


def _sc_impl(x_hbm, src_hbm, dst_hbm, agg_out, cnt_out, zbuf, rows, sidx,
             didx, ones, zvec, agg_sh, cnt_sh, sem, with_cnt):
    cid = lax.axis_index("c")
    sid = lax.axis_index("s")
    wid = sid * NC + cid

    def _zero_row(i, _):
        r = i // (D // 16)
        c = i % (D // 16)
        zbuf[r, pl.ds(c * 16, 16)] = jnp.zeros((16,), jnp.float32)
        return 0

    lax.fori_loop(0, CHUNK * (D // 16), _zero_row, 0)
    for k in range(RPT // CHUNK):
        pltpu.sync_copy(zbuf.at[pl.ds(0, CHUNK)],
                        agg_sh.at[pl.ds(sid * RPT + k * CHUNK, CHUNK)])
    if with_cnt:
        def _init_ones(i, _):
            ones[pl.ds(i * 16, 16)] = jnp.ones((16,), jnp.float32)
            return 0

        lax.fori_loop(0, CHUNK // 16, _init_ones, 0)

        def _zero_zvec(i, _):
            zvec[pl.ds(i * 16, 16)] = jnp.zeros((16,), jnp.float32)
            return 0

        lax.fori_loop(0, RPT // 16, _zero_zvec, 0)
        pltpu.sync_copy(zvec, cnt_sh.at[pl.ds(sid * RPT, RPT)])

    plsc.subcore_barrier()

    def _edge_chunk(i, _):
        base = wid * EPW + i * CHUNK
        pltpu.sync_copy(src_hbm.at[pl.ds(base, CHUNK)], sidx)
        pltpu.sync_copy(dst_hbm.at[pl.ds(base, CHUNK)], didx)
        pltpu.async_copy(x_hbm.at[sidx], rows, sem).wait()
        pltpu.sync_copy(rows, agg_sh.at[didx], add=True)
        if with_cnt:
            pltpu.sync_copy(ones, cnt_sh.at[didx], add=True)
        return 0

    lax.fori_loop(0, NCHUNK, _edge_chunk, 0)
    plsc.subcore_barrier()

    for k in range(RPT // CHUNK):
        r0 = sid * RPT + k * CHUNK
        pltpu.sync_copy(agg_sh.at[pl.ds(r0, CHUNK)],
                        agg_out.at[cid, pl.ds(r0, CHUNK)])
    if with_cnt:
        pltpu.sync_copy(cnt_sh.at[pl.ds(sid * RPT, RPT)],
                        cnt_out.at[cid, pl.ds(sid * RPT, RPT)])


def _sc_body_cnt(x_hbm, src_hbm, dst_hbm, agg_out, cnt_out, zbuf, rows,
                 sidx, didx, ones, zvec, agg_sh, cnt_sh, sem):
    _sc_impl(x_hbm, src_hbm, dst_hbm, agg_out, cnt_out, zbuf, rows, sidx,
             didx, ones, zvec, agg_sh, cnt_sh, sem, True)


def _sc_body_plain(x_hbm, src_hbm, dst_hbm, agg_out, zbuf, rows, sidx,
                   didx, agg_sh, sem):
    _sc_impl(x_hbm, src_hbm, dst_hbm, agg_out, None, zbuf, rows, sidx,
             didx, None, None, agg_sh, None, sem, False)


_MESH = plsc.VectorSubcoreMesh(core_axis_name="c", subcore_axis_name="s")

_sc_agg_cnt = pl.kernel(
    _sc_body_cnt,
    out_type=(jax.ShapeDtypeStruct((NC, NP, D), jnp.float32),
              jax.ShapeDtypeStruct((NC, NP), jnp.float32)),
    mesh=_MESH,
    scratch_types=[
        pltpu.VMEM((CHUNK, D), jnp.float32),
        pltpu.VMEM((CHUNK, D), jnp.float32),
        pltpu.VMEM((CHUNK,), jnp.int32),
        pltpu.VMEM((CHUNK,), jnp.int32),
        pltpu.VMEM((CHUNK,), jnp.float32),
        pltpu.VMEM((RPT,), jnp.float32),
        pltpu.VMEM_SHARED((NP, D), jnp.float32),
        pltpu.VMEM_SHARED((NP,), jnp.float32),
        pltpu.SemaphoreType.DMA,
    ],
    name="sage_sc_agg_cnt",
)

_sc_agg = pl.kernel(
    _sc_body_plain,
    out_type=jax.ShapeDtypeStruct((NC, NP, D), jnp.float32),
    mesh=_MESH,
    scratch_types=[
        pltpu.VMEM((CHUNK, D), jnp.float32),
        pltpu.VMEM((CHUNK, D), jnp.float32),
        pltpu.VMEM((CHUNK,), jnp.int32),
        pltpu.VMEM((CHUNK,), jnp.int32),
        pltpu.VMEM_SHARED((NP, D), jnp.float32),
        pltpu.SemaphoreType.DMA,
    ],
    name="sage_sc_agg",
)

ROWS_BLK = 1024


def _dense_body(agg_ref, cnt_ref, x_ref, wl_ref, bl_ref, wr_ref, out_ref,
                *, relu):
    agg = agg_ref[0] + agg_ref[1]
    cnt = (cnt_ref[0] + cnt_ref[1])[:, None]
    mean = agg / jnp.maximum(cnt, 1.0)
    acc = (jnp.dot(mean, wl_ref[...], preferred_element_type=jnp.float32)
           + jnp.dot(x_ref[...], wr_ref[...],
                     preferred_element_type=jnp.float32)
           + bl_ref[...])
    if relu:
        acc = jnp.maximum(acc, 0.0)
    out_ref[...] = acc


def _dense(aggp, cntp, x, wl_t, bl, wr_t, relu):
    grid = (NP // ROWS_BLK,)
    return pl.pallas_call(
        functools.partial(_dense_body, relu=relu),
        grid=grid,
        in_specs=[
            pl.BlockSpec((NC, ROWS_BLK, D), lambda i: (0, i, 0)),
            pl.BlockSpec((NC, ROWS_BLK), lambda i: (0, i)),
            pl.BlockSpec((ROWS_BLK, D), lambda i: (i, 0)),
            pl.BlockSpec((D, D), lambda i: (0, 0)),
            pl.BlockSpec((1, D), lambda i: (0, 0)),
            pl.BlockSpec((D, D), lambda i: (0, 0)),
        ],
        out_specs=pl.BlockSpec((ROWS_BLK, D), lambda i: (i, 0)),
        out_shape=jax.ShapeDtypeStruct((NP, D), jnp.float32),
    )(aggp, cntp, x, wl_t, bl, wr_t)


def kernel(x, edge_index, W1_l, b1_l, W1_r, W2_l, b2_l, W2_r):
    ei = edge_index.astype(jnp.int32)
    src = ei[0]
    dst = ei[1]
    xp = jnp.pad(x, ((0, NP - N), (0, 0)))
    aggp1, cntp = _sc_agg_cnt(xp, src, dst)
    h = _dense(aggp1, cntp, xp, W1_l.T, b1_l.reshape(1, D), W1_r.T,
               relu=True)
    aggp2 = _sc_agg(h, src, dst)
    out = _dense(aggp2, cntp, h, W2_l.T, b2_l.reshape(1, D), W2_r.T,
                 relu=False)
    return out[:N]


# speedup vs baseline: 9.4554x; 1.8477x over previous
import functools

import jax
import jax.numpy as jnp
from jax import lax
from jax.experimental import pallas as pl
from jax.experimental.pallas import tpu as pltpu
from jax.experimental.pallas import tpu_sc as plsc

N = 10000
E = 320000
D = 128

_INFO = plsc.get_sparse_core_info()
NC = _INFO.num_cores
NS = _INFO.num_subcores
NW = NC * NS
NP = 10240
EPW = E // NW
CHUNK = 80
NCHUNK = EPW // CHUNK
RPT = NP // NS


def _sc_impl(x_hbm, src_hbm, dst_hbm, agg_out, cnt_out, rows_a, rows_b,
             sidx, didx, ones, zvec, agg_sh, cnt_sh, sem_a, sem_b,
             with_cnt):
    cid = lax.axis_index("c")
    sid = lax.axis_index("s")
    wid = sid * NC + cid

    def _zero_row(i, _):
        r = i // (D // 16)
        c = i % (D // 16)
        rows_a[r, pl.ds(c * 16, 16)] = jnp.zeros((16,), jnp.float32)
        return 0

    lax.fori_loop(0, CHUNK * (D // 16), _zero_row, 0)
    for k in range(RPT // CHUNK):
        pltpu.sync_copy(rows_a.at[pl.ds(0, CHUNK)],
                        agg_sh.at[pl.ds(sid * RPT + k * CHUNK, CHUNK)])
    base = wid * EPW

    pltpu.async_copy(src_hbm.at[pl.ds(base, EPW)], sidx, sem_a)

    def _idx_fire(j, _):
        pltpu.async_copy(dst_hbm.at[pl.ds(base + j * CHUNK, CHUNK)],
                         didx.at[j], sem_b)
        return 0

    lax.fori_loop(0, NCHUNK, _idx_fire, 0)
    pltpu.make_async_copy(src_hbm.at[pl.ds(base, EPW)], sidx, sem_a).wait()

    def _idx_drain(j, _):
        pltpu.make_async_copy(dst_hbm.at[pl.ds(base + j * CHUNK, CHUNK)],
                              didx.at[j], sem_b).wait()
        return 0

    lax.fori_loop(0, NCHUNK, _idx_drain, 0)
    if with_cnt:
        def _init_ones(i, _):
            ones[pl.ds(i * 16, 16)] = jnp.ones((16,), jnp.float32)
            return 0

        lax.fori_loop(0, CHUNK // 16, _init_ones, 0)

        def _zero_zvec(i, _):
            zvec[pl.ds(i * 16, 16)] = jnp.zeros((16,), jnp.float32)
            return 0

        lax.fori_loop(0, RPT // 16, _zero_zvec, 0)
        pltpu.sync_copy(zvec, cnt_sh.at[pl.ds(sid * RPT, RPT)])

    plsc.subcore_barrier()

    def _scatter(rows, j):
        pltpu.sync_copy(rows, agg_sh.at[didx.at[j]], add=True)
        if with_cnt:
            pltpu.sync_copy(ones, cnt_sh.at[didx.at[j]], add=True)

    def _sl(j):
        return sidx.at[pl.ds(j * CHUNK, CHUNK)]

    pltpu.async_copy(x_hbm.at[_sl(0)], rows_a, sem_a)

    def _edge_pair(i, _):
        ca = 2 * i
        pltpu.make_async_copy(x_hbm.at[_sl(ca)], rows_a, sem_a).wait()
        pltpu.async_copy(x_hbm.at[_sl(ca + 1)], rows_b, sem_b)
        _scatter(rows_a, ca)
        pltpu.make_async_copy(x_hbm.at[_sl(ca + 1)], rows_b, sem_b).wait()
        pltpu.async_copy(x_hbm.at[_sl(ca + 2)], rows_a, sem_a)
        _scatter(rows_b, ca + 1)
        return 0

    lax.fori_loop(0, (NCHUNK - 1) // 2, _edge_pair, 0)
    pltpu.make_async_copy(x_hbm.at[_sl(NCHUNK - 1)], rows_a, sem_a).wait()
    _scatter(rows_a, NCHUNK - 1)
    plsc.subcore_barrier()

    for k in range(RPT // CHUNK):
        r0 = sid * RPT + k * CHUNK
        pltpu.sync_copy(agg_sh.at[pl.ds(r0, CHUNK)],
                        agg_out.at[cid, pl.ds(r0, CHUNK)])
    if with_cnt:
        pltpu.sync_copy(cnt_sh.at[pl.ds(sid * RPT, RPT)],
                        cnt_out.at[cid, pl.ds(sid * RPT, RPT)])


def _sc_body_cnt(x_hbm, src_hbm, dst_hbm, agg_out, cnt_out, rows_a,
                 rows_b, sidx, didx, ones, zvec, agg_sh, cnt_sh, sem_a,
                 sem_b):
    _sc_impl(x_hbm, src_hbm, dst_hbm, agg_out, cnt_out, rows_a, rows_b,
             sidx, didx, ones, zvec, agg_sh, cnt_sh, sem_a, sem_b, True)


def _sc_body_plain(x_hbm, src_hbm, dst_hbm, agg_out, rows_a, rows_b,
                   sidx, didx, agg_sh, sem_a, sem_b):
    _sc_impl(x_hbm, src_hbm, dst_hbm, agg_out, None, rows_a, rows_b,
             sidx, didx, None, None, agg_sh, None, sem_a, sem_b, False)


_MESH = plsc.VectorSubcoreMesh(core_axis_name="c", subcore_axis_name="s")

_sc_agg_cnt = pl.kernel(
    _sc_body_cnt,
    out_type=(jax.ShapeDtypeStruct((NC, NP, D), jnp.float32),
              jax.ShapeDtypeStruct((NC, NP), jnp.float32)),
    mesh=_MESH,
    scratch_types=[
        pltpu.VMEM((CHUNK, D), jnp.float32),
        pltpu.VMEM((CHUNK, D), jnp.float32),
        pltpu.VMEM((EPW,), jnp.int32),
        pltpu.VMEM((NCHUNK, CHUNK), jnp.int32),
        pltpu.VMEM((CHUNK,), jnp.float32),
        pltpu.VMEM((RPT,), jnp.float32),
        pltpu.VMEM_SHARED((NP, D), jnp.float32),
        pltpu.VMEM_SHARED((NP,), jnp.float32),
        pltpu.SemaphoreType.DMA,
        pltpu.SemaphoreType.DMA,
    ],
    name="sage_sc_agg_cnt",
)

_sc_agg = pl.kernel(
    _sc_body_plain,
    out_type=jax.ShapeDtypeStruct((NC, NP, D), jnp.float32),
    mesh=_MESH,
    scratch_types=[
        pltpu.VMEM((CHUNK, D), jnp.float32),
        pltpu.VMEM((CHUNK, D), jnp.float32),
        pltpu.VMEM((EPW,), jnp.int32),
        pltpu.VMEM((NCHUNK, CHUNK), jnp.int32),
        pltpu.VMEM_SHARED((NP, D), jnp.float32),
        pltpu.SemaphoreType.DMA,
        pltpu.SemaphoreType.DMA,
    ],
    name="sage_sc_agg",
)

ROWS_BLK = 1024


def _dense_body(agg_ref, cnt_ref, x_ref, wl_ref, bl_ref, wr_ref, out_ref,
                *, relu):
    agg = agg_ref[0] + agg_ref[1]
    cnt = (cnt_ref[0] + cnt_ref[1])[:, None]
    mean = agg / jnp.maximum(cnt, 1.0)
    acc = (jnp.dot(mean, wl_ref[...], preferred_element_type=jnp.float32)
           + jnp.dot(x_ref[...], wr_ref[...],
                     preferred_element_type=jnp.float32)
           + bl_ref[...])
    if relu:
        acc = jnp.maximum(acc, 0.0)
    out_ref[...] = acc


def _dense(aggp, cntp, x, wl_t, bl, wr_t, relu):
    grid = (NP // ROWS_BLK,)
    return pl.pallas_call(
        functools.partial(_dense_body, relu=relu),
        grid=grid,
        in_specs=[
            pl.BlockSpec((NC, ROWS_BLK, D), lambda i: (0, i, 0)),
            pl.BlockSpec((NC, ROWS_BLK), lambda i: (0, i)),
            pl.BlockSpec((ROWS_BLK, D), lambda i: (i, 0)),
            pl.BlockSpec((D, D), lambda i: (0, 0)),
            pl.BlockSpec((1, D), lambda i: (0, 0)),
            pl.BlockSpec((D, D), lambda i: (0, 0)),
        ],
        out_specs=pl.BlockSpec((ROWS_BLK, D), lambda i: (i, 0)),
        out_shape=jax.ShapeDtypeStruct((NP, D), jnp.float32),
    )(aggp, cntp, x, wl_t, bl, wr_t)


def kernel(x, edge_index, W1_l, b1_l, W1_r, W2_l, b2_l, W2_r):
    ei = edge_index.astype(jnp.int32)
    src = ei[0]
    dst = ei[1]
    xp = jnp.pad(x, ((0, NP - N), (0, 0)))
    aggp1, cntp = _sc_agg_cnt(xp, src, dst)
    h = _dense(aggp1, cntp, xp, W1_l.T, b1_l.reshape(1, D), W1_r.T,
               relu=True)
    aggp2 = _sc_agg(h, src, dst)
    out = _dense(aggp2, cntp, h, W2_l.T, b2_l.reshape(1, D), W2_r.T,
                 relu=False)
    return out[:N]


# speedup vs baseline: 9.9597x; 1.0533x over previous
import functools

import jax
import jax.numpy as jnp
from jax import lax
from jax.experimental import pallas as pl
from jax.experimental.pallas import tpu as pltpu
from jax.experimental.pallas import tpu_sc as plsc

N = 10000
E = 320000
D = 128

_INFO = plsc.get_sparse_core_info()
NC = _INFO.num_cores
NS = _INFO.num_subcores
NW = NC * NS
NP = 10240
CHUNK = 128
EPW = 10240
EP = NW * EPW
NCHUNK = EPW // CHUNK
RPT = NP // NS


def _sc_impl(x_hbm, src_hbm, dst_hbm, agg_out, cnt_out, rows_a, rows_b,
             sidx_a, sidx_b, didx, ones, zvec, agg_sh, cnt_sh, sem_ga,
             sem_gb, sem_ia, sem_ib, sem_sa, sem_sb, with_cnt):
    cid = lax.axis_index("c")
    sid = lax.axis_index("s")
    wid = sid * NC + cid
    base = wid * EPW

    def _zero_row(i, _):
        r = i // (D // 16)
        c = i % (D // 16)
        rows_a[r, pl.ds(c * 16, 16)] = jnp.zeros((16,), jnp.float32)
        return 0

    lax.fori_loop(0, CHUNK * (D // 16), _zero_row, 0)
    for k in range(RPT // CHUNK):
        pltpu.sync_copy(rows_a.at[pl.ds(0, CHUNK)],
                        agg_sh.at[pl.ds(sid * RPT + k * CHUNK, CHUNK)])

    def _idx_fire(j, _):
        pltpu.async_copy(dst_hbm.at[pl.ds(base + j * CHUNK, CHUNK)],
                         didx.at[j], sem_ia)
        return 0

    lax.fori_loop(0, NCHUNK, _idx_fire, 0)

    def _idx_drain(j, _):
        pltpu.make_async_copy(dst_hbm.at[pl.ds(base + j * CHUNK, CHUNK)],
                              didx.at[j], sem_ia).wait()
        return 0

    lax.fori_loop(0, NCHUNK, _idx_drain, 0)
    if with_cnt:
        def _init_ones(i, _):
            ones[pl.ds(i * 16, 16)] = jnp.ones((16,), jnp.float32)
            return 0

        lax.fori_loop(0, CHUNK // 16, _init_ones, 0)

        def _zero_zvec(i, _):
            zvec[pl.ds(i * 16, 16)] = jnp.zeros((16,), jnp.float32)
            return 0

        lax.fori_loop(0, RPT // 16, _zero_zvec, 0)
        pltpu.sync_copy(zvec, cnt_sh.at[pl.ds(sid * RPT, RPT)])

    plsc.subcore_barrier()

    def _src_dma(c, ref, sem):
        return pltpu.async_copy(src_hbm.at[pl.ds(base + c * CHUNK, CHUNK)],
                                ref, sem)

    def _src_wait(c, ref, sem):
        pltpu.make_async_copy(src_hbm.at[pl.ds(base + c * CHUNK, CHUNK)],
                              ref, sem).wait()

    def _scat_start(rows, c, sem):
        pltpu.async_copy(rows, agg_sh.at[didx.at[c]], sem, add=True)
        if with_cnt:
            pltpu.async_copy(ones, cnt_sh.at[didx.at[c]], sem, add=True)

    def _scat_wait(rows, c, sem):
        pltpu.make_async_copy(rows, agg_sh.at[didx.at[c]], sem).wait()
        if with_cnt:
            pltpu.make_async_copy(ones, cnt_sh.at[didx.at[c]],
                                  sem).wait()

    _src_dma(0, sidx_a, sem_ia)
    _src_wait(0, sidx_a, sem_ia)
    pltpu.async_copy(x_hbm.at[sidx_a], rows_a, sem_ga)
    _src_dma(1, sidx_b, sem_ib)
    _src_wait(1, sidx_b, sem_ib)
    pltpu.async_copy(x_hbm.at[sidx_b], rows_b, sem_gb)

    def _edge_pair(i, _):
        c = 2 * i
        pltpu.make_async_copy(x_hbm.at[sidx_a], rows_a, sem_ga).wait()
        _scat_start(rows_a, c, sem_sa)
        _src_dma(c + 2, sidx_a, sem_ia)
        pltpu.make_async_copy(x_hbm.at[sidx_b], rows_b, sem_gb).wait()
        _scat_start(rows_b, c + 1, sem_sb)
        _src_dma(c + 3, sidx_b, sem_ib)
        _scat_wait(rows_a, c, sem_sa)
        _src_wait(c + 2, sidx_a, sem_ia)
        pltpu.async_copy(x_hbm.at[sidx_a], rows_a, sem_ga)
        _scat_wait(rows_b, c + 1, sem_sb)
        _src_wait(c + 3, sidx_b, sem_ib)
        pltpu.async_copy(x_hbm.at[sidx_b], rows_b, sem_gb)
        return 0

    lax.fori_loop(0, NCHUNK // 2 - 1, _edge_pair, 0)
    c_last = NCHUNK - 2
    pltpu.make_async_copy(x_hbm.at[sidx_a], rows_a, sem_ga).wait()
    _scat_start(rows_a, c_last, sem_sa)
    pltpu.make_async_copy(x_hbm.at[sidx_b], rows_b, sem_gb).wait()
    _scat_start(rows_b, c_last + 1, sem_sb)
    _scat_wait(rows_a, c_last, sem_sa)
    _scat_wait(rows_b, c_last + 1, sem_sb)
    plsc.subcore_barrier()

    for k in range(RPT // CHUNK):
        r0 = sid * RPT + k * CHUNK
        pltpu.sync_copy(agg_sh.at[pl.ds(r0, CHUNK)],
                        agg_out.at[cid, pl.ds(r0, CHUNK)])
    if with_cnt:
        pltpu.sync_copy(cnt_sh.at[pl.ds(sid * RPT, RPT)],
                        cnt_out.at[cid, pl.ds(sid * RPT, RPT)])


def _sc_body_cnt(x_hbm, src_hbm, dst_hbm, agg_out, cnt_out, rows_a,
                 rows_b, sidx_a, sidx_b, didx, ones, zvec, agg_sh, cnt_sh,
                 sem_ga, sem_gb, sem_ia, sem_ib, sem_sa, sem_sb):
    _sc_impl(x_hbm, src_hbm, dst_hbm, agg_out, cnt_out, rows_a, rows_b,
             sidx_a, sidx_b, didx, ones, zvec, agg_sh, cnt_sh, sem_ga,
             sem_gb, sem_ia, sem_ib, sem_sa, sem_sb, True)


def _sc_body_plain(x_hbm, src_hbm, dst_hbm, agg_out, rows_a, rows_b,
                   sidx_a, sidx_b, didx, agg_sh, sem_ga, sem_gb, sem_ia,
                   sem_ib, sem_sa, sem_sb):
    _sc_impl(x_hbm, src_hbm, dst_hbm, agg_out, None, rows_a, rows_b,
             sidx_a, sidx_b, didx, None, None, agg_sh, None, sem_ga,
             sem_gb, sem_ia, sem_ib, sem_sa, sem_sb, False)


_MESH = plsc.VectorSubcoreMesh(core_axis_name="c", subcore_axis_name="s")

_sc_agg_cnt = pl.kernel(
    _sc_body_cnt,
    out_type=(jax.ShapeDtypeStruct((NC, NP, D), jnp.float32),
              jax.ShapeDtypeStruct((NC, NP), jnp.float32)),
    mesh=_MESH,
    scratch_types=[
        pltpu.VMEM((CHUNK, D), jnp.float32),
        pltpu.VMEM((CHUNK, D), jnp.float32),
        pltpu.VMEM((CHUNK,), jnp.int32),
        pltpu.VMEM((CHUNK,), jnp.int32),
        pltpu.VMEM((NCHUNK, CHUNK), jnp.int32),
        pltpu.VMEM((CHUNK,), jnp.float32),
        pltpu.VMEM((RPT,), jnp.float32),
        pltpu.VMEM_SHARED((NP, D), jnp.float32),
        pltpu.VMEM_SHARED((NP,), jnp.float32),
        pltpu.SemaphoreType.DMA,
        pltpu.SemaphoreType.DMA,
        pltpu.SemaphoreType.DMA,
        pltpu.SemaphoreType.DMA,
        pltpu.SemaphoreType.DMA,
        pltpu.SemaphoreType.DMA,
    ],
    name="sage_sc_agg_cnt",
)

_sc_agg = pl.kernel(
    _sc_body_plain,
    out_type=jax.ShapeDtypeStruct((NC, NP, D), jnp.float32),
    mesh=_MESH,
    scratch_types=[
        pltpu.VMEM((CHUNK, D), jnp.float32),
        pltpu.VMEM((CHUNK, D), jnp.float32),
        pltpu.VMEM((CHUNK,), jnp.int32),
        pltpu.VMEM((CHUNK,), jnp.int32),
        pltpu.VMEM((NCHUNK, CHUNK), jnp.int32),
        pltpu.VMEM_SHARED((NP, D), jnp.float32),
        pltpu.SemaphoreType.DMA,
        pltpu.SemaphoreType.DMA,
        pltpu.SemaphoreType.DMA,
        pltpu.SemaphoreType.DMA,
        pltpu.SemaphoreType.DMA,
        pltpu.SemaphoreType.DMA,
    ],
    name="sage_sc_agg",
)

ROWS_BLK = 1024


def _dense_body(agg_ref, cnt_ref, x_ref, wl_ref, bl_ref, wr_ref, out_ref,
                *, relu):
    agg = agg_ref[0] + agg_ref[1]
    cnt = (cnt_ref[0] + cnt_ref[1])[:, None]
    mean = agg / jnp.maximum(cnt, 1.0)
    acc = (jnp.dot(mean, wl_ref[...], preferred_element_type=jnp.float32)
           + jnp.dot(x_ref[...], wr_ref[...],
                     preferred_element_type=jnp.float32)
           + bl_ref[...])
    if relu:
        acc = jnp.maximum(acc, 0.0)
    out_ref[...] = acc


def _dense(aggp, cntp, x, wl_t, bl, wr_t, relu):
    grid = (NP // ROWS_BLK,)
    return pl.pallas_call(
        functools.partial(_dense_body, relu=relu),
        grid=grid,
        in_specs=[
            pl.BlockSpec((NC, ROWS_BLK, D), lambda i: (0, i, 0)),
            pl.BlockSpec((NC, ROWS_BLK), lambda i: (0, i)),
            pl.BlockSpec((ROWS_BLK, D), lambda i: (i, 0)),
            pl.BlockSpec((D, D), lambda i: (0, 0)),
            pl.BlockSpec((1, D), lambda i: (0, 0)),
            pl.BlockSpec((D, D), lambda i: (0, 0)),
        ],
        out_specs=pl.BlockSpec((ROWS_BLK, D), lambda i: (i, 0)),
        out_shape=jax.ShapeDtypeStruct((NP, D), jnp.float32),
    )(aggp, cntp, x, wl_t, bl, wr_t)


def kernel(x, edge_index, W1_l, b1_l, W1_r, W2_l, b2_l, W2_r):
    ei = edge_index.astype(jnp.int32)
    npad = EP - E
    pad_iota = jnp.arange(npad, dtype=jnp.int32)
    src = jnp.concatenate([ei[0], pad_iota % N])
    dst = jnp.concatenate([ei[1], N + pad_iota % (NP - N)])
    xp = jnp.pad(x, ((0, NP - N), (0, 0)))
    aggp1, cntp = _sc_agg_cnt(xp, src, dst)
    h = _dense(aggp1, cntp, xp, W1_l.T, b1_l.reshape(1, D), W1_r.T,
               relu=True)
    aggp2 = _sc_agg(h, src, dst)
    out = _dense(aggp2, cntp, h, W2_l.T, b2_l.reshape(1, D), W2_r.T,
                 relu=False)
    return out[:N]
